# baseline (speedup 1.0000x reference)
import functools

import jax
import jax.numpy as jnp
from jax import lax
from jax.experimental import pallas as pl
from jax.experimental.pallas import tpu as pltpu
from jax.experimental.pallas import tpu_sc as plsc

N_NODES = 10000
NACC = 10240
HIDDEN = 128
GEO_PAD = 128
EPS = 1e-5
BLK = 256
CH = 128
NW = 32


def _sc_redistribute(nf, geo_pad, i_o, j_o, k_o, slot):
    n = i_o.shape[0]
    P = n + 4 * BLK
    nchunk = n // CH
    steps = (nchunk + NW - 1) // NW
    mesh = plsc.VectorSubcoreMesh(core_axis_name="c", subcore_axis_name="s")

    @functools.partial(
        pl.kernel,
        out_type=jax.ShapeDtypeStruct((4, P, HIDDEN), jnp.float32),
        mesh=mesh,
        scratch_types=[
            pltpu.VMEM((CH,), jnp.int32),
            pltpu.VMEM((CH,), jnp.int32),
            pltpu.VMEM((CH, HIDDEN), jnp.float32),
            pltpu.VMEM((CH, GEO_PAD), jnp.float32),
            pltpu.SemaphoreType.DMA,
        ],
    )
    def k(nf_h, geo_h, i_h, j_h, k_h, slot_h, g_h,
          idx_v, slot_v, buf_v, gbuf_v, sem):
        wid = lax.axis_index("s") * 2 + lax.axis_index("c")

        def body(t, carry):
            g = wid + t * NW

            @pl.when(g < nchunk)
            def _():
                base = g * CH
                pltpu.sync_copy(slot_h.at[pl.ds(base, CH)], slot_v)
                for a, ih in enumerate((i_h, j_h, k_h)):
                    pltpu.sync_copy(ih.at[pl.ds(base, CH)], idx_v)
                    pltpu.async_copy(nf_h.at[idx_v], buf_v, sem).wait()
                    pltpu.async_copy(buf_v, g_h.at[a].at[slot_v], sem).wait()
                pltpu.sync_copy(geo_h.at[pl.ds(base, CH)], gbuf_v)
                pltpu.async_copy(gbuf_v, g_h.at[3].at[slot_v], sem).wait()

            return carry

        lax.fori_loop(0, steps, body, 0)

    return k(nf, geo_pad, i_o, j_o, k_o, slot)


def _sc_scatter(z, i_o, slot, zeros_init):
    n = i_o.shape[0]
    nchunk = n // CH
    half0 = (nchunk + 1) // 2
    steps = (half0 + 15) // 16
    rows_per_tile = NACC // 16
    mesh = plsc.VectorSubcoreMesh(core_axis_name="c", subcore_axis_name="s")

    @functools.partial(
        pl.kernel,
        out_type=jax.ShapeDtypeStruct((2, NACC, HIDDEN), jnp.float32),
        mesh=mesh,
        scratch_types=[
            pltpu.VMEM((CH,), jnp.int32),
            pltpu.VMEM((CH,), jnp.int32),
            pltpu.VMEM((CH, HIDDEN), jnp.float32),
            pltpu.VMEM_SHARED((NACC, HIDDEN), jnp.float32),
            pltpu.SemaphoreType.DMA,
        ],
    )
    def k(z_h, i_h, slot_h, zero_h, out_h, idx_v, slot_v, z_v, acc, sem):
        c = lax.axis_index("c")
        s = lax.axis_index("s")

        @pl.when(s == 0)
        def _():
            pltpu.sync_copy(zero_h, acc)

        plsc.subcore_barrier()
        start_c = c * half0
        end_c = half0 + c * (nchunk - half0)

        def body(t, carry):
            g = start_c + s + t * 16

            @pl.when(g < end_c)
            def _():
                base = g * CH
                pltpu.sync_copy(slot_h.at[pl.ds(base, CH)], slot_v)
                pltpu.async_copy(z_h.at[slot_v], z_v, sem).wait()
                pltpu.sync_copy(i_h.at[pl.ds(base, CH)], idx_v)
                pltpu.sync_copy(z_v, acc.at[idx_v], add=True)

            return carry

        lax.fori_loop(0, steps, body, 0)
        plsc.subcore_barrier()
        r0 = s * rows_per_tile
        pltpu.sync_copy(acc.at[pl.ds(r0, rows_per_tile)],
                        out_h.at[c, pl.ds(r0, rows_per_tile)])

    return k(z, i_o, slot, zeros_init)


def _valid_mask(m, br, ve_ref):
    rows = m * BLK + lax.broadcasted_iota(jnp.int32, (BLK, 1), 0)
    return rows < ve_ref[br]


def _accum_stats(m, br, y, v, s_ref, q_ref):
    @pl.when(m == 0)
    def _():
        s_ref[...] = jnp.zeros_like(s_ref)
        q_ref[...] = jnp.zeros_like(q_ref)

    s_ref[br] += jnp.sum(jnp.where(v, y, 0.0), axis=0, keepdims=True)
    q_ref[br] += jnp.sum(jnp.where(v, y * y, 0.0), axis=0, keepdims=True)


def _l0_body(bb_ref, ve_ref, cnt_ref, g0, g1, g2, g3, wa, wb, wc, wg, b0,
             y_ref, s_ref, q_ref):
    m = pl.program_id(0)
    br = bb_ref[m]
    y = jnp.dot(g0[0], wa[br], preferred_element_type=jnp.float32)
    y += jnp.dot(g1[0], wb[br], preferred_element_type=jnp.float32)
    y += jnp.dot(g2[0], wc[br], preferred_element_type=jnp.float32)
    y += jnp.dot(g3[0], wg[br], preferred_element_type=jnp.float32)
    y += b0[br]
    y_ref[...] = y
    _accum_stats(m, br, y, _valid_mask(m, br, ve_ref), s_ref, q_ref)


def _fused_body(bb_ref, ve_ref, cnt_ref, y0, s0, q0, wh, bh3, ga4, be4, attb,
                gin, y_out, s_sc, q_sc):
    p = pl.program_id(0)
    m = pl.program_id(1)
    br = bb_ref[m]
    cnt = cnt_ref[br]
    pc = jnp.bitwise_and(p, 1)
    pp = jnp.bitwise_and(p - 1, 1)
    prev_s = jnp.where(p == 0, s0[br], s_sc[pp, br])
    prev_q = jnp.where(p == 0, q0[br], q_sc[pp, br])
    mean = prev_s / cnt
    var = prev_q / cnt - mean * mean
    yprev = jnp.where(p == 0, y0[...], gin[0])
    x = (yprev - mean) * lax.rsqrt(var + EPS)
    x = x * ga4[br, p] + be4[br, p]
    x = jnp.maximum(x, 0.0)
    rows = m * BLK + lax.broadcasted_iota(jnp.int32, (BLK, 1), 0)
    vmask = rows < ve_ref[br]

    @pl.when(p < 3)
    def _():
        li = jnp.minimum(p, 2)
        y = jnp.dot(x, wh[br, li], preferred_element_type=jnp.float32)
        y += bh3[br, li]
        y_out[0] = y

        @pl.when(m == 0)
        def _():
            s_sc[pc] = jnp.zeros_like(s0)
            q_sc[pc] = jnp.zeros_like(q0)

        s_sc[pc, br] += jnp.sum(jnp.where(vmask, y, 0.0), axis=0, keepdims=True)
        q_sc[pc, br] += jnp.sum(jnp.where(vmask, y * y, 0.0), axis=0, keepdims=True)

    @pl.when(p == 3)
    def _():
        y_out[0] = jnp.where(vmask, x * attb[br], 0.0)


def _full(shape):
    nd = len(shape)
    return pl.BlockSpec(shape, lambda m, *_: (0,) * nd)


def _rows(width):
    return pl.BlockSpec((BLK, width), lambda m, *_: (m, 0))


def _tc_call(body, grid, in_specs, out_specs, out_shape, args,
             scratch_shapes=(), aliases=()):
    grid_spec = pltpu.PrefetchScalarGridSpec(
        num_scalar_prefetch=3,
        grid=grid,
        in_specs=in_specs,
        out_specs=out_specs,
        scratch_shapes=list(scratch_shapes),
    )
    return pl.pallas_call(
        body, grid_spec=grid_spec, out_shape=out_shape,
        input_output_aliases=dict(aliases),
        compiler_params=pltpu.CompilerParams(
            dimension_semantics=("arbitrary",) * len(grid)),
    )(*args)


def _add_body(a, b, o):
    o[...] = a[...] + b[...]


def kernel(node_feature, geo_encoding, edge_index_2rd, edx_jk, edx_ij,
           num_edge_inside, att, g, W0, b0, Wh, bh, gamma, beta):
    n = edx_ij.shape[0]
    P = n + 4 * BLK
    nb = P // BLK

    i = edge_index_2rd[0]
    j = edge_index_2rd[1]
    kk = edge_index_2rd[2]
    br = 2 * (edx_ij >= num_edge_inside).astype(jnp.int32) \
        + (edx_jk >= num_edge_inside).astype(jnp.int32)

    onehot = (br[:, None] == jnp.arange(4)[None, :]).astype(jnp.int32)
    counts = jnp.sum(onehot, axis=0)
    padded = ((counts + BLK - 1) // BLK) * BLK
    pstart = jnp.concatenate([jnp.zeros(1, jnp.int32),
                              jnp.cumsum(padded)[:3].astype(jnp.int32)])
    rank = jnp.sum(jnp.cumsum(onehot, axis=0) * onehot, axis=1) - 1
    slot = (pstart[br] + rank).astype(jnp.int32)
    valid_end = pstart + counts
    block_starts = jnp.arange(nb, dtype=jnp.int32) * BLK
    block_branch = jnp.sum(
        (block_starts[:, None] >= pstart[None, 1:]).astype(jnp.int32), axis=1)
    cnt_f = counts.astype(jnp.float32)

    geo_pad = jnp.pad(geo_encoding, ((0, 0), (0, GEO_PAD - geo_encoding.shape[1])))
    Wa = W0[:, 0:HIDDEN]
    Wb = W0[:, HIDDEN:2 * HIDDEN]
    Wc = W0[:, 2 * HIDDEN:3 * HIDDEN]
    Wg = jnp.pad(W0[:, 3 * HIDDEN:], ((0, 0), (0, GEO_PAD - (W0.shape[1] - 3 * HIDDEN)), (0, 0)))
    att_b = jnp.broadcast_to(att[:, None, None], (4, 1, HIDDEN))

    G = _sc_redistribute(node_feature, geo_pad,
                         i.astype(jnp.int32),
                         j.astype(jnp.int32),
                         kk.astype(jnp.int32), slot)

    stats_shape = jax.ShapeDtypeStruct((4, 1, HIDDEN), jnp.float32)
    y_shape = jax.ShapeDtypeStruct((P, HIDDEN), jnp.float32)
    scalars = (block_branch, valid_end, cnt_f)

    def _gview(a):
        return pl.BlockSpec((1, BLK, HIDDEN), lambda m, *_, _a=a: (_a, m, 0))

    y0, s0, q0 = _tc_call(
        _l0_body, (nb,),
        [_gview(0), _gview(1), _gview(2), _gview(3),
         _full((4, HIDDEN, HIDDEN)), _full((4, HIDDEN, HIDDEN)),
         _full((4, HIDDEN, HIDDEN)), _full((4, GEO_PAD, HIDDEN)),
         _full((4, 1, HIDDEN))],
        [_rows(HIDDEN), _full((4, 1, HIDDEN)), _full((4, 1, HIDDEN))],
        [y_shape, stats_shape, stats_shape],
        (*scalars, G, G, G, G, Wa, Wb, Wc, Wg, b0[:, None, :]),
    )

    gamma4 = gamma[:, :, None, :]
    beta4 = beta[:, :, None, :]
    bh3 = bh[:, :, None, :]
    y0_spec = pl.BlockSpec((BLK, HIDDEN),
                           lambda p, m, *_: (jnp.where(p == 0, m, 0), 0))
    yin_spec = pl.BlockSpec(
        (1, BLK, HIDDEN),
        lambda p, m, *_: (jnp.where(p == 0, 3, p - 1),
                          jnp.where(p == 0, 0, m), 0))
    yout_spec = pl.BlockSpec((1, BLK, HIDDEN), lambda p, m, *_: (p, m, 0))
    Y = _tc_call(
        _fused_body, (4, nb),
        [y0_spec, _full((4, 1, HIDDEN)), _full((4, 1, HIDDEN)),
         _full((4, 3, HIDDEN, HIDDEN)), _full((4, 3, 1, HIDDEN)),
         _full((4, 4, 1, HIDDEN)), _full((4, 4, 1, HIDDEN)),
         _full((4, 1, HIDDEN)), yin_spec],
        [yout_spec],
        [jax.ShapeDtypeStruct((4, P, HIDDEN), jnp.float32)],
        (*scalars, y0, s0, q0, Wh, bh3, gamma4, beta4, att_b, G),
        scratch_shapes=[pltpu.VMEM((2, 4, 1, HIDDEN), jnp.float32),
                        pltpu.VMEM((2, 4, 1, HIDDEN), jnp.float32)],
        aliases=((11, 0),),
    )[0]

    zeros_init = jnp.zeros((NACC, HIDDEN), jnp.float32)
    zflat = Y.reshape(4 * P, HIDDEN)
    slot3 = slot + 3 * P
    partials = _sc_scatter(zflat, i.astype(jnp.int32), slot3, zeros_init)

    out = pl.pallas_call(
        _add_body,
        grid=(N_NODES // 400,),
        in_specs=[pl.BlockSpec((400, HIDDEN), lambda m: (m, 0)),
                  pl.BlockSpec((400, HIDDEN), lambda m: (m, 0))],
        out_specs=pl.BlockSpec((400, HIDDEN), lambda m: (m, 0)),
        out_shape=jax.ShapeDtypeStruct((N_NODES, HIDDEN), jnp.float32),
    )(partials[0], partials[1])
    return out


# speedup vs baseline: 1.4617x; 1.4617x over previous
import functools

import jax
import jax.numpy as jnp
from jax import lax
from jax.experimental import pallas as pl
from jax.experimental.pallas import tpu as pltpu
from jax.experimental.pallas import tpu_sc as plsc

N_NODES = 10000
NACC = 10240
HIDDEN = 128
GEO_PAD = 128
EPS = 1e-5
BLK = 512
CH = 128
NW = 32


def _sc_redistribute(nf, geo_pad, i_o, j_o, k_o, slot):
    n = i_o.shape[0]
    P = ((n + 4 * BLK) + BLK - 1) // BLK * BLK
    nchunk = n // CH
    steps = (nchunk + NW - 1) // NW
    mesh = plsc.VectorSubcoreMesh(core_axis_name="c", subcore_axis_name="s")

    @functools.partial(
        pl.kernel,
        out_type=[
            jax.ShapeDtypeStruct((P, HIDDEN), jnp.float32),
            jax.ShapeDtypeStruct((P, HIDDEN), jnp.float32),
            jax.ShapeDtypeStruct((P, HIDDEN), jnp.float32),
            jax.ShapeDtypeStruct((P, GEO_PAD), jnp.float32),
        ],
        mesh=mesh,
        scratch_types=[
            pltpu.VMEM((CH,), jnp.int32),
            pltpu.VMEM((CH,), jnp.int32),
            pltpu.VMEM((CH, HIDDEN), jnp.float32),
            pltpu.VMEM((CH, GEO_PAD), jnp.float32),
            pltpu.SemaphoreType.DMA,
        ],
    )
    def k(nf_h, geo_h, i_h, j_h, k_h, slot_h, xi_h, xj_h, xk_h, geo_o,
          idx_v, slot_v, buf_v, gbuf_v, sem):
        wid = lax.axis_index("s") * 2 + lax.axis_index("c")

        def body(t, carry):
            g = wid + t * NW

            @pl.when(g < nchunk)
            def _():
                base = g * CH
                pltpu.sync_copy(slot_h.at[pl.ds(base, CH)], slot_v)
                for ih, oh in ((i_h, xi_h), (j_h, xj_h), (k_h, xk_h)):
                    pltpu.sync_copy(ih.at[pl.ds(base, CH)], idx_v)
                    pltpu.async_copy(nf_h.at[idx_v], buf_v, sem).wait()
                    pltpu.async_copy(buf_v, oh.at[slot_v], sem).wait()
                pltpu.sync_copy(geo_h.at[pl.ds(base, CH)], gbuf_v)
                pltpu.async_copy(gbuf_v, geo_o.at[slot_v], sem).wait()

            return carry

        lax.fori_loop(0, steps, body, 0)

    return k(nf, geo_pad, i_o, j_o, k_o, slot)


def _sc_scatter(z, i_o, slot, zeros_init):
    n = i_o.shape[0]
    nchunk = n // CH
    half0 = (nchunk + 1) // 2
    steps = (half0 + 15) // 16
    rows_per_tile = NACC // 16
    mesh = plsc.VectorSubcoreMesh(core_axis_name="c", subcore_axis_name="s")

    @functools.partial(
        pl.kernel,
        out_type=jax.ShapeDtypeStruct((2, NACC, HIDDEN), jnp.float32),
        mesh=mesh,
        scratch_types=[
            pltpu.VMEM((CH,), jnp.int32),
            pltpu.VMEM((CH,), jnp.int32),
            pltpu.VMEM((CH, HIDDEN), jnp.float32),
            pltpu.VMEM_SHARED((NACC, HIDDEN), jnp.float32),
            pltpu.SemaphoreType.DMA,
        ],
    )
    def k(z_h, i_h, slot_h, zero_h, out_h, idx_v, slot_v, z_v, acc, sem):
        c = lax.axis_index("c")
        s = lax.axis_index("s")

        @pl.when(s == 0)
        def _():
            pltpu.sync_copy(zero_h, acc)

        plsc.subcore_barrier()
        start_c = c * half0
        end_c = half0 + c * (nchunk - half0)

        def body(t, carry):
            g = start_c + s + t * 16

            @pl.when(g < end_c)
            def _():
                base = g * CH
                pltpu.sync_copy(slot_h.at[pl.ds(base, CH)], slot_v)
                pltpu.async_copy(z_h.at[slot_v], z_v, sem).wait()
                pltpu.sync_copy(i_h.at[pl.ds(base, CH)], idx_v)
                pltpu.sync_copy(z_v, acc.at[idx_v], add=True)

            return carry

        lax.fori_loop(0, steps, body, 0)
        plsc.subcore_barrier()
        r0 = s * rows_per_tile
        pltpu.sync_copy(acc.at[pl.ds(r0, rows_per_tile)],
                        out_h.at[c, pl.ds(r0, rows_per_tile)])

    return k(z, i_o, slot, zeros_init)


def _valid_mask(m, br, ve_ref):
    rows = m * BLK + lax.broadcasted_iota(jnp.int32, (BLK, 1), 0)
    return rows < ve_ref[br]


def _accum_stats(m, br, y, v, s_ref, q_ref):
    @pl.when(m == 0)
    def _():
        s_ref[...] = jnp.zeros_like(s_ref)
        q_ref[...] = jnp.zeros_like(q_ref)

    s_ref[br] += jnp.sum(jnp.where(v, y, 0.0), axis=0, keepdims=True)
    q_ref[br] += jnp.sum(jnp.where(v, y * y, 0.0), axis=0, keepdims=True)


def _l0_body(bb_ref, ve_ref, cnt_ref, xi, xj, xk, gg, wa, wb, wc, wg, b0,
             y_ref, s_ref, q_ref):
    m = pl.program_id(0)
    br = bb_ref[m]
    y = jnp.dot(xi[...], wa[br], preferred_element_type=jnp.float32)
    y += jnp.dot(xj[...], wb[br], preferred_element_type=jnp.float32)
    y += jnp.dot(xk[...], wc[br], preferred_element_type=jnp.float32)
    y += jnp.dot(gg[...], wg[br], preferred_element_type=jnp.float32)
    y += b0[br]
    y_ref[...] = y
    _accum_stats(m, br, y, _valid_mask(m, br, ve_ref), s_ref, q_ref)


def _mid_body(bb_ref, ve_ref, cnt_ref, yp, sp, qp, w, bv, ga, be,
              y_ref, s_ref, q_ref):
    m = pl.program_id(0)
    br = bb_ref[m]
    cnt = cnt_ref[br]
    mean = sp[br] / cnt
    var = qp[br] / cnt - mean * mean
    x = (yp[...] - mean) * lax.rsqrt(var + EPS)
    x = x * ga[br] + be[br]
    x = jnp.maximum(x, 0.0)
    y = jnp.dot(x, w[br], preferred_element_type=jnp.float32) + bv[br]
    y_ref[...] = y
    _accum_stats(m, br, y, _valid_mask(m, br, ve_ref), s_ref, q_ref)


def _fin_body(bb_ref, ve_ref, cnt_ref, yp, sp, qp, ga, be, attb, z_ref):
    m = pl.program_id(0)
    br = bb_ref[m]
    cnt = cnt_ref[br]
    mean = sp[br] / cnt
    var = qp[br] / cnt - mean * mean
    x = (yp[...] - mean) * lax.rsqrt(var + EPS)
    x = x * ga[br] + be[br]
    x = jnp.maximum(x, 0.0)
    z_ref[...] = jnp.where(_valid_mask(m, br, ve_ref), x * attb[br], 0.0)


def _full(shape):
    nd = len(shape)
    return pl.BlockSpec(shape, lambda m, *_: (0,) * nd)


def _rows(width):
    return pl.BlockSpec((BLK, width), lambda m, *_: (m, 0))


def _tc_call(body, nb, in_specs, out_specs, out_shape, args):
    grid_spec = pltpu.PrefetchScalarGridSpec(
        num_scalar_prefetch=3,
        grid=(nb,),
        in_specs=in_specs,
        out_specs=out_specs,
    )
    return pl.pallas_call(body, grid_spec=grid_spec, out_shape=out_shape)(*args)


def _add_body(a, b, o):
    o[...] = a[...] + b[...]


def kernel(node_feature, geo_encoding, edge_index_2rd, edx_jk, edx_ij,
           num_edge_inside, att, g, W0, b0, Wh, bh, gamma, beta):
    n = edx_ij.shape[0]
    P = ((n + 4 * BLK) + BLK - 1) // BLK * BLK
    nb = P // BLK

    i = edge_index_2rd[0]
    j = edge_index_2rd[1]
    kk = edge_index_2rd[2]
    br = 2 * (edx_ij >= num_edge_inside).astype(jnp.int32) \
        + (edx_jk >= num_edge_inside).astype(jnp.int32)

    onehot = (br[:, None] == jnp.arange(4)[None, :]).astype(jnp.int32)
    counts = jnp.sum(onehot, axis=0)
    padded = ((counts + BLK - 1) // BLK) * BLK
    pstart = jnp.concatenate([jnp.zeros(1, jnp.int32),
                              jnp.cumsum(padded)[:3].astype(jnp.int32)])
    rank = jnp.sum(jnp.cumsum(onehot, axis=0) * onehot, axis=1) - 1
    slot = (pstart[br] + rank).astype(jnp.int32)
    valid_end = pstart + counts
    block_starts = jnp.arange(nb, dtype=jnp.int32) * BLK
    block_branch = jnp.sum(
        (block_starts[:, None] >= pstart[None, 1:]).astype(jnp.int32), axis=1)
    cnt_f = counts.astype(jnp.float32)

    geo_pad = jnp.pad(geo_encoding, ((0, 0), (0, GEO_PAD - geo_encoding.shape[1])))
    Wa = W0[:, 0:HIDDEN]
    Wb = W0[:, HIDDEN:2 * HIDDEN]
    Wc = W0[:, 2 * HIDDEN:3 * HIDDEN]
    Wg = jnp.pad(W0[:, 3 * HIDDEN:], ((0, 0), (0, GEO_PAD - (W0.shape[1] - 3 * HIDDEN)), (0, 0)))
    att_b = jnp.broadcast_to(att[:, None, None], (4, 1, HIDDEN))

    xi, xj, xk, geo_s = _sc_redistribute(node_feature, geo_pad,
                                         i.astype(jnp.int32),
                                         j.astype(jnp.int32),
                                         kk.astype(jnp.int32), slot)

    stats_shape = jax.ShapeDtypeStruct((4, 1, HIDDEN), jnp.float32)
    y_shape = jax.ShapeDtypeStruct((P, HIDDEN), jnp.float32)
    scalars = (block_branch, valid_end, cnt_f)

    y, s, q = _tc_call(
        _l0_body, nb,
        [_rows(HIDDEN), _rows(HIDDEN), _rows(HIDDEN), _rows(GEO_PAD),
         _full((4, HIDDEN, HIDDEN)), _full((4, HIDDEN, HIDDEN)),
         _full((4, HIDDEN, HIDDEN)), _full((4, GEO_PAD, HIDDEN)),
         _full((4, 1, HIDDEN))],
        [_rows(HIDDEN), _full((4, 1, HIDDEN)), _full((4, 1, HIDDEN))],
        [y_shape, stats_shape, stats_shape],
        (*scalars, xi, xj, xk, geo_s, Wa, Wb, Wc, Wg, b0[:, None, :]),
    )

    for layer in range(1, 4):
        y, s, q = _tc_call(
            _mid_body, nb,
            [_rows(HIDDEN), _full((4, 1, HIDDEN)), _full((4, 1, HIDDEN)),
             _full((4, HIDDEN, HIDDEN)), _full((4, 1, HIDDEN)),
             _full((4, 1, HIDDEN)), _full((4, 1, HIDDEN))],
            [_rows(HIDDEN), _full((4, 1, HIDDEN)), _full((4, 1, HIDDEN))],
            [y_shape, stats_shape, stats_shape],
            (*scalars, y, s, q, Wh[:, layer - 1], bh[:, layer - 1, :, None].transpose(0, 2, 1),
             gamma[:, layer - 1][:, None, :], beta[:, layer - 1][:, None, :]),
        )

    z = _tc_call(
        _fin_body, nb,
        [_rows(HIDDEN), _full((4, 1, HIDDEN)), _full((4, 1, HIDDEN)),
         _full((4, 1, HIDDEN)), _full((4, 1, HIDDEN)), _full((4, 1, HIDDEN))],
        [_rows(HIDDEN)],
        [y_shape],
        (*scalars, y, s, q, gamma[:, 3][:, None, :], beta[:, 3][:, None, :], att_b),
    )[0]

    zeros_init = jnp.zeros((NACC, HIDDEN), jnp.float32)
    partials = _sc_scatter(z, i.astype(jnp.int32), slot, zeros_init)

    out = pl.pallas_call(
        _add_body,
        grid=(N_NODES // 400,),
        in_specs=[pl.BlockSpec((400, HIDDEN), lambda m: (m, 0)),
                  pl.BlockSpec((400, HIDDEN), lambda m: (m, 0))],
        out_specs=pl.BlockSpec((400, HIDDEN), lambda m: (m, 0)),
        out_shape=jax.ShapeDtypeStruct((N_NODES, HIDDEN), jnp.float32),
    )(partials[0], partials[1])
    return out


# speedup vs baseline: 1.8488x; 1.2648x over previous
import functools

import jax
import jax.numpy as jnp
from jax import lax
from jax.experimental import pallas as pl
from jax.experimental.pallas import tpu as pltpu
from jax.experimental.pallas import tpu_sc as plsc

N_NODES = 10000
NACC = 10240
HIDDEN = 128
GEO_PAD = 128
EPS = 1e-5
BLK = 1024
CH = 128
NW = 32


def _sc_redistribute(nf, geo_pad, i_o, j_o, k_o, slot):
    n = i_o.shape[0]
    P = ((n + 4 * BLK) + BLK - 1) // BLK * BLK
    nchunk = n // CH
    steps = (nchunk + NW - 1) // NW
    mesh = plsc.VectorSubcoreMesh(core_axis_name="c", subcore_axis_name="s")

    @functools.partial(
        pl.kernel,
        out_type=[
            jax.ShapeDtypeStruct((P, HIDDEN), jnp.float32),
            jax.ShapeDtypeStruct((P, HIDDEN), jnp.float32),
            jax.ShapeDtypeStruct((P, HIDDEN), jnp.float32),
            jax.ShapeDtypeStruct((P, GEO_PAD), jnp.float32),
        ],
        mesh=mesh,
        scratch_types=[
            pltpu.VMEM((CH,), jnp.int32),
            pltpu.VMEM((CH,), jnp.int32),
            pltpu.VMEM((CH, HIDDEN), jnp.float32),
            pltpu.VMEM((CH, GEO_PAD), jnp.float32),
            pltpu.SemaphoreType.DMA,
        ],
    )
    def k(nf_h, geo_h, i_h, j_h, k_h, slot_h, xi_h, xj_h, xk_h, geo_o,
          idx_v, slot_v, buf_v, gbuf_v, sem):
        wid = lax.axis_index("s") * 2 + lax.axis_index("c")

        def body(t, carry):
            g = wid + t * NW

            @pl.when(g < nchunk)
            def _():
                base = g * CH
                pltpu.sync_copy(slot_h.at[pl.ds(base, CH)], slot_v)
                for ih, oh in ((i_h, xi_h), (j_h, xj_h), (k_h, xk_h)):
                    pltpu.sync_copy(ih.at[pl.ds(base, CH)], idx_v)
                    pltpu.async_copy(nf_h.at[idx_v], buf_v, sem).wait()
                    pltpu.async_copy(buf_v, oh.at[slot_v], sem).wait()
                pltpu.sync_copy(geo_h.at[pl.ds(base, CH)], gbuf_v)
                pltpu.async_copy(gbuf_v, geo_o.at[slot_v], sem).wait()

            return carry

        lax.fori_loop(0, steps, body, 0)

    return k(nf, geo_pad, i_o, j_o, k_o, slot)


def _sc_scatter(z, i_o, slot, zeros_init):
    n = i_o.shape[0]
    nchunk = n // CH
    half0 = (nchunk + 1) // 2
    steps = (half0 + 15) // 16
    rows_per_tile = NACC // 16
    mesh = plsc.VectorSubcoreMesh(core_axis_name="c", subcore_axis_name="s")

    @functools.partial(
        pl.kernel,
        out_type=jax.ShapeDtypeStruct((2, NACC, HIDDEN), jnp.float32),
        mesh=mesh,
        scratch_types=[
            pltpu.VMEM((CH,), jnp.int32),
            pltpu.VMEM((CH,), jnp.int32),
            pltpu.VMEM((CH, HIDDEN), jnp.float32),
            pltpu.VMEM_SHARED((NACC, HIDDEN), jnp.float32),
            pltpu.SemaphoreType.DMA,
        ],
    )
    def k(z_h, i_h, slot_h, zero_h, out_h, idx_v, slot_v, z_v, acc, sem):
        c = lax.axis_index("c")
        s = lax.axis_index("s")

        @pl.when(s == 0)
        def _():
            pltpu.sync_copy(zero_h, acc)

        plsc.subcore_barrier()
        start_c = c * half0
        end_c = half0 + c * (nchunk - half0)

        def body(t, carry):
            g = start_c + s + t * 16

            @pl.when(g < end_c)
            def _():
                base = g * CH
                pltpu.sync_copy(slot_h.at[pl.ds(base, CH)], slot_v)
                pltpu.async_copy(z_h.at[slot_v], z_v, sem).wait()
                pltpu.sync_copy(i_h.at[pl.ds(base, CH)], idx_v)
                pltpu.sync_copy(z_v, acc.at[idx_v], add=True)

            return carry

        lax.fori_loop(0, steps, body, 0)
        plsc.subcore_barrier()
        r0 = s * rows_per_tile
        pltpu.sync_copy(acc.at[pl.ds(r0, rows_per_tile)],
                        out_h.at[c, pl.ds(r0, rows_per_tile)])

    return k(z, i_o, slot, zeros_init)


def _valid_mask(m, br, ve_ref):
    rows = m * BLK + lax.broadcasted_iota(jnp.int32, (BLK, 1), 0)
    return rows < ve_ref[br]


def _accum_stats(m, br, y, v, s_ref, q_ref):
    @pl.when(m == 0)
    def _():
        s_ref[...] = jnp.zeros_like(s_ref)
        q_ref[...] = jnp.zeros_like(q_ref)

    s_ref[br] += jnp.sum(jnp.where(v, y, 0.0), axis=0, keepdims=True)
    q_ref[br] += jnp.sum(jnp.where(v, y * y, 0.0), axis=0, keepdims=True)


def _l0_body(bb_ref, ve_ref, cnt_ref, xi, xj, xk, gg, wa, wb, wc, wg, b0,
             y_ref, s_ref, q_ref):
    m = pl.program_id(0)
    br = bb_ref[m]
    y = jnp.dot(xi[...], wa[br], preferred_element_type=jnp.float32)
    y += jnp.dot(xj[...], wb[br], preferred_element_type=jnp.float32)
    y += jnp.dot(xk[...], wc[br], preferred_element_type=jnp.float32)
    y += jnp.dot(gg[...], wg[br], preferred_element_type=jnp.float32)
    y += b0[br]
    y_ref[...] = y
    _accum_stats(m, br, y, _valid_mask(m, br, ve_ref), s_ref, q_ref)


def _mid_body(bb_ref, ve_ref, cnt_ref, yp, sp, qp, w, bv, ga, be,
              y_ref, s_ref, q_ref):
    m = pl.program_id(0)
    br = bb_ref[m]
    cnt = cnt_ref[br]
    mean = sp[br] / cnt
    var = qp[br] / cnt - mean * mean
    x = (yp[...] - mean) * lax.rsqrt(var + EPS)
    x = x * ga[br] + be[br]
    x = jnp.maximum(x, 0.0)
    y = jnp.dot(x, w[br], preferred_element_type=jnp.float32) + bv[br]
    y_ref[...] = y
    _accum_stats(m, br, y, _valid_mask(m, br, ve_ref), s_ref, q_ref)


def _fin_body(bb_ref, ve_ref, cnt_ref, yp, sp, qp, ga, be, attb, z_ref):
    m = pl.program_id(0)
    br = bb_ref[m]
    cnt = cnt_ref[br]
    mean = sp[br] / cnt
    var = qp[br] / cnt - mean * mean
    x = (yp[...] - mean) * lax.rsqrt(var + EPS)
    x = x * ga[br] + be[br]
    x = jnp.maximum(x, 0.0)
    z_ref[...] = jnp.where(_valid_mask(m, br, ve_ref), x * attb[br], 0.0)


def _full(shape):
    nd = len(shape)
    return pl.BlockSpec(shape, lambda m, *_: (0,) * nd)


def _rows(width):
    return pl.BlockSpec((BLK, width), lambda m, *_: (m, 0))


def _tc_call(body, nb, in_specs, out_specs, out_shape, args):
    grid_spec = pltpu.PrefetchScalarGridSpec(
        num_scalar_prefetch=3,
        grid=(nb,),
        in_specs=in_specs,
        out_specs=out_specs,
    )
    return pl.pallas_call(body, grid_spec=grid_spec, out_shape=out_shape)(*args)


def _add_body(a, b, o):
    o[...] = a[...] + b[...]


def kernel(node_feature, geo_encoding, edge_index_2rd, edx_jk, edx_ij,
           num_edge_inside, att, g, W0, b0, Wh, bh, gamma, beta):
    n = edx_ij.shape[0]
    P = ((n + 4 * BLK) + BLK - 1) // BLK * BLK
    nb = P // BLK

    i = edge_index_2rd[0]
    j = edge_index_2rd[1]
    kk = edge_index_2rd[2]
    br = 2 * (edx_ij >= num_edge_inside).astype(jnp.int32) \
        + (edx_jk >= num_edge_inside).astype(jnp.int32)

    onehot = (br[:, None] == jnp.arange(4)[None, :]).astype(jnp.int32)
    counts = jnp.sum(onehot, axis=0)
    padded = ((counts + BLK - 1) // BLK) * BLK
    pstart = jnp.concatenate([jnp.zeros(1, jnp.int32),
                              jnp.cumsum(padded)[:3].astype(jnp.int32)])
    rank = jnp.sum(jnp.cumsum(onehot, axis=0) * onehot, axis=1) - 1
    slot = (pstart[br] + rank).astype(jnp.int32)
    valid_end = pstart + counts
    block_starts = jnp.arange(nb, dtype=jnp.int32) * BLK
    block_branch = jnp.sum(
        (block_starts[:, None] >= pstart[None, 1:]).astype(jnp.int32), axis=1)
    cnt_f = counts.astype(jnp.float32)

    geo_pad = jnp.pad(geo_encoding, ((0, 0), (0, GEO_PAD - geo_encoding.shape[1])))
    Wa = W0[:, 0:HIDDEN]
    Wb = W0[:, HIDDEN:2 * HIDDEN]
    Wc = W0[:, 2 * HIDDEN:3 * HIDDEN]
    Wg = jnp.pad(W0[:, 3 * HIDDEN:], ((0, 0), (0, GEO_PAD - (W0.shape[1] - 3 * HIDDEN)), (0, 0)))
    att_b = jnp.broadcast_to(att[:, None, None], (4, 1, HIDDEN))

    xi, xj, xk, geo_s = _sc_redistribute(node_feature, geo_pad,
                                         i.astype(jnp.int32),
                                         j.astype(jnp.int32),
                                         kk.astype(jnp.int32), slot)

    stats_shape = jax.ShapeDtypeStruct((4, 1, HIDDEN), jnp.float32)
    y_shape = jax.ShapeDtypeStruct((P, HIDDEN), jnp.float32)
    scalars = (block_branch, valid_end, cnt_f)

    y, s, q = _tc_call(
        _l0_body, nb,
        [_rows(HIDDEN), _rows(HIDDEN), _rows(HIDDEN), _rows(GEO_PAD),
         _full((4, HIDDEN, HIDDEN)), _full((4, HIDDEN, HIDDEN)),
         _full((4, HIDDEN, HIDDEN)), _full((4, GEO_PAD, HIDDEN)),
         _full((4, 1, HIDDEN))],
        [_rows(HIDDEN), _full((4, 1, HIDDEN)), _full((4, 1, HIDDEN))],
        [y_shape, stats_shape, stats_shape],
        (*scalars, xi, xj, xk, geo_s, Wa, Wb, Wc, Wg, b0[:, None, :]),
    )

    for layer in range(1, 4):
        y, s, q = _tc_call(
            _mid_body, nb,
            [_rows(HIDDEN), _full((4, 1, HIDDEN)), _full((4, 1, HIDDEN)),
             _full((4, HIDDEN, HIDDEN)), _full((4, 1, HIDDEN)),
             _full((4, 1, HIDDEN)), _full((4, 1, HIDDEN))],
            [_rows(HIDDEN), _full((4, 1, HIDDEN)), _full((4, 1, HIDDEN))],
            [y_shape, stats_shape, stats_shape],
            (*scalars, y, s, q, Wh[:, layer - 1], bh[:, layer - 1, :, None].transpose(0, 2, 1),
             gamma[:, layer - 1][:, None, :], beta[:, layer - 1][:, None, :]),
        )

    z = _tc_call(
        _fin_body, nb,
        [_rows(HIDDEN), _full((4, 1, HIDDEN)), _full((4, 1, HIDDEN)),
         _full((4, 1, HIDDEN)), _full((4, 1, HIDDEN)), _full((4, 1, HIDDEN))],
        [_rows(HIDDEN)],
        [y_shape],
        (*scalars, y, s, q, gamma[:, 3][:, None, :], beta[:, 3][:, None, :], att_b),
    )[0]

    zeros_init = jnp.zeros((NACC, HIDDEN), jnp.float32)
    partials = _sc_scatter(z, i.astype(jnp.int32), slot, zeros_init)

    out = pl.pallas_call(
        _add_body,
        grid=(N_NODES // 400,),
        in_specs=[pl.BlockSpec((400, HIDDEN), lambda m: (m, 0)),
                  pl.BlockSpec((400, HIDDEN), lambda m: (m, 0))],
        out_specs=pl.BlockSpec((400, HIDDEN), lambda m: (m, 0)),
        out_shape=jax.ShapeDtypeStruct((N_NODES, HIDDEN), jnp.float32),
    )(partials[0], partials[1])
    return out


# speedup vs baseline: 2.1409x; 1.1580x over previous
import functools

import jax
import jax.numpy as jnp
from jax import lax
from jax.experimental import pallas as pl
from jax.experimental.pallas import tpu as pltpu
from jax.experimental.pallas import tpu_sc as plsc

N_NODES = 10000
NACC = 10240
HIDDEN = 128
GEO_PAD = 128
EPS = 1e-5
BLK = 2048
CH = 128
NW = 32


def _sc_redistribute(nf, geo_pad, i_o, j_o, k_o, slot):
    n = i_o.shape[0]
    P = ((n + 4 * BLK) + BLK - 1) // BLK * BLK
    nchunk = n // CH
    steps = (nchunk + NW - 1) // NW
    mesh = plsc.VectorSubcoreMesh(core_axis_name="c", subcore_axis_name="s")

    @functools.partial(
        pl.kernel,
        out_type=[
            jax.ShapeDtypeStruct((P, HIDDEN), jnp.float32),
            jax.ShapeDtypeStruct((P, HIDDEN), jnp.float32),
            jax.ShapeDtypeStruct((P, HIDDEN), jnp.float32),
            jax.ShapeDtypeStruct((P, GEO_PAD), jnp.float32),
        ],
        mesh=mesh,
        scratch_types=[
            pltpu.VMEM((CH,), jnp.int32),
            pltpu.VMEM((CH,), jnp.int32),
            pltpu.VMEM((CH, HIDDEN), jnp.float32),
            pltpu.VMEM((CH, GEO_PAD), jnp.float32),
            pltpu.SemaphoreType.DMA,
        ],
    )
    def k(nf_h, geo_h, i_h, j_h, k_h, slot_h, xi_h, xj_h, xk_h, geo_o,
          idx_v, slot_v, buf_v, gbuf_v, sem):
        wid = lax.axis_index("s") * 2 + lax.axis_index("c")

        def body(t, carry):
            g = wid + t * NW

            @pl.when(g < nchunk)
            def _():
                base = g * CH
                pltpu.sync_copy(slot_h.at[pl.ds(base, CH)], slot_v)
                for ih, oh in ((i_h, xi_h), (j_h, xj_h), (k_h, xk_h)):
                    pltpu.sync_copy(ih.at[pl.ds(base, CH)], idx_v)
                    pltpu.async_copy(nf_h.at[idx_v], buf_v, sem).wait()
                    pltpu.async_copy(buf_v, oh.at[slot_v], sem).wait()
                pltpu.sync_copy(geo_h.at[pl.ds(base, CH)], gbuf_v)
                pltpu.async_copy(gbuf_v, geo_o.at[slot_v], sem).wait()

            return carry

        lax.fori_loop(0, steps, body, 0)

    return k(nf, geo_pad, i_o, j_o, k_o, slot)


def _sc_scatter(z, i_o, slot, zeros_init):
    n = i_o.shape[0]
    nchunk = n // CH
    half0 = (nchunk + 1) // 2
    steps = (half0 + 15) // 16
    rows_per_tile = NACC // 16
    mesh = plsc.VectorSubcoreMesh(core_axis_name="c", subcore_axis_name="s")

    @functools.partial(
        pl.kernel,
        out_type=jax.ShapeDtypeStruct((2, NACC, HIDDEN), jnp.float32),
        mesh=mesh,
        scratch_types=[
            pltpu.VMEM((CH,), jnp.int32),
            pltpu.VMEM((CH,), jnp.int32),
            pltpu.VMEM((CH, HIDDEN), jnp.float32),
            pltpu.VMEM_SHARED((NACC, HIDDEN), jnp.float32),
            pltpu.SemaphoreType.DMA,
        ],
    )
    def k(z_h, i_h, slot_h, zero_h, out_h, idx_v, slot_v, z_v, acc, sem):
        c = lax.axis_index("c")
        s = lax.axis_index("s")

        @pl.when(s == 0)
        def _():
            pltpu.sync_copy(zero_h, acc)

        plsc.subcore_barrier()
        start_c = c * half0
        end_c = half0 + c * (nchunk - half0)

        def body(t, carry):
            g = start_c + s + t * 16

            @pl.when(g < end_c)
            def _():
                base = g * CH
                pltpu.sync_copy(slot_h.at[pl.ds(base, CH)], slot_v)
                pltpu.async_copy(z_h.at[slot_v], z_v, sem).wait()
                pltpu.sync_copy(i_h.at[pl.ds(base, CH)], idx_v)
                pltpu.sync_copy(z_v, acc.at[idx_v], add=True)

            return carry

        lax.fori_loop(0, steps, body, 0)
        plsc.subcore_barrier()
        r0 = s * rows_per_tile
        pltpu.sync_copy(acc.at[pl.ds(r0, rows_per_tile)],
                        out_h.at[c, pl.ds(r0, rows_per_tile)])

    return k(z, i_o, slot, zeros_init)


def _valid_mask(m, br, ve_ref):
    rows = m * BLK + lax.broadcasted_iota(jnp.int32, (BLK, 1), 0)
    return rows < ve_ref[br]


def _accum_stats(m, br, y, v, s_ref, q_ref):
    @pl.when(m == 0)
    def _():
        s_ref[...] = jnp.zeros_like(s_ref)
        q_ref[...] = jnp.zeros_like(q_ref)

    s_ref[br] += jnp.sum(jnp.where(v, y, 0.0), axis=0, keepdims=True)
    q_ref[br] += jnp.sum(jnp.where(v, y * y, 0.0), axis=0, keepdims=True)


def _l0_body(bb_ref, ve_ref, cnt_ref, xi, xj, xk, gg, wa, wb, wc, wg, b0,
             y_ref, s_ref, q_ref):
    m = pl.program_id(0)
    br = bb_ref[m]
    y = jnp.dot(xi[...], wa[br], preferred_element_type=jnp.float32)
    y += jnp.dot(xj[...], wb[br], preferred_element_type=jnp.float32)
    y += jnp.dot(xk[...], wc[br], preferred_element_type=jnp.float32)
    y += jnp.dot(gg[...], wg[br], preferred_element_type=jnp.float32)
    y += b0[br]
    y_ref[...] = y
    _accum_stats(m, br, y, _valid_mask(m, br, ve_ref), s_ref, q_ref)


def _mid_body(bb_ref, ve_ref, cnt_ref, yp, sp, qp, w, bv, ga, be,
              y_ref, s_ref, q_ref):
    m = pl.program_id(0)
    br = bb_ref[m]
    cnt = cnt_ref[br]
    mean = sp[br] / cnt
    var = qp[br] / cnt - mean * mean
    x = (yp[...] - mean) * lax.rsqrt(var + EPS)
    x = x * ga[br] + be[br]
    x = jnp.maximum(x, 0.0)
    y = jnp.dot(x, w[br], preferred_element_type=jnp.float32) + bv[br]
    y_ref[...] = y
    _accum_stats(m, br, y, _valid_mask(m, br, ve_ref), s_ref, q_ref)


def _fin_body(bb_ref, ve_ref, cnt_ref, yp, sp, qp, ga, be, attb, z_ref):
    m = pl.program_id(0)
    br = bb_ref[m]
    cnt = cnt_ref[br]
    mean = sp[br] / cnt
    var = qp[br] / cnt - mean * mean
    x = (yp[...] - mean) * lax.rsqrt(var + EPS)
    x = x * ga[br] + be[br]
    x = jnp.maximum(x, 0.0)
    z_ref[...] = jnp.where(_valid_mask(m, br, ve_ref), x * attb[br], 0.0)


def _full(shape):
    nd = len(shape)
    return pl.BlockSpec(shape, lambda m, *_: (0,) * nd)


def _rows(width):
    return pl.BlockSpec((BLK, width), lambda m, *_: (m, 0))


def _tc_call(body, nb, in_specs, out_specs, out_shape, args):
    grid_spec = pltpu.PrefetchScalarGridSpec(
        num_scalar_prefetch=3,
        grid=(nb,),
        in_specs=in_specs,
        out_specs=out_specs,
    )
    return pl.pallas_call(body, grid_spec=grid_spec, out_shape=out_shape)(*args)


def _add_body(a, b, o):
    o[...] = a[...] + b[...]


def kernel(node_feature, geo_encoding, edge_index_2rd, edx_jk, edx_ij,
           num_edge_inside, att, g, W0, b0, Wh, bh, gamma, beta):
    n = edx_ij.shape[0]
    P = ((n + 4 * BLK) + BLK - 1) // BLK * BLK
    nb = P // BLK

    i = edge_index_2rd[0]
    j = edge_index_2rd[1]
    kk = edge_index_2rd[2]
    br = 2 * (edx_ij >= num_edge_inside).astype(jnp.int32) \
        + (edx_jk >= num_edge_inside).astype(jnp.int32)

    onehot = (br[:, None] == jnp.arange(4)[None, :]).astype(jnp.int32)
    counts = jnp.sum(onehot, axis=0)
    padded = ((counts + BLK - 1) // BLK) * BLK
    pstart = jnp.concatenate([jnp.zeros(1, jnp.int32),
                              jnp.cumsum(padded)[:3].astype(jnp.int32)])
    rank = jnp.sum(jnp.cumsum(onehot, axis=0) * onehot, axis=1) - 1
    slot = (pstart[br] + rank).astype(jnp.int32)
    valid_end = pstart + counts
    block_starts = jnp.arange(nb, dtype=jnp.int32) * BLK
    block_branch = jnp.sum(
        (block_starts[:, None] >= pstart[None, 1:]).astype(jnp.int32), axis=1)
    cnt_f = counts.astype(jnp.float32)

    geo_pad = jnp.pad(geo_encoding, ((0, 0), (0, GEO_PAD - geo_encoding.shape[1])))
    Wa = W0[:, 0:HIDDEN]
    Wb = W0[:, HIDDEN:2 * HIDDEN]
    Wc = W0[:, 2 * HIDDEN:3 * HIDDEN]
    Wg = jnp.pad(W0[:, 3 * HIDDEN:], ((0, 0), (0, GEO_PAD - (W0.shape[1] - 3 * HIDDEN)), (0, 0)))
    att_b = jnp.broadcast_to(att[:, None, None], (4, 1, HIDDEN))

    xi, xj, xk, geo_s = _sc_redistribute(node_feature, geo_pad,
                                         i.astype(jnp.int32),
                                         j.astype(jnp.int32),
                                         kk.astype(jnp.int32), slot)

    stats_shape = jax.ShapeDtypeStruct((4, 1, HIDDEN), jnp.float32)
    y_shape = jax.ShapeDtypeStruct((P, HIDDEN), jnp.float32)
    scalars = (block_branch, valid_end, cnt_f)

    y, s, q = _tc_call(
        _l0_body, nb,
        [_rows(HIDDEN), _rows(HIDDEN), _rows(HIDDEN), _rows(GEO_PAD),
         _full((4, HIDDEN, HIDDEN)), _full((4, HIDDEN, HIDDEN)),
         _full((4, HIDDEN, HIDDEN)), _full((4, GEO_PAD, HIDDEN)),
         _full((4, 1, HIDDEN))],
        [_rows(HIDDEN), _full((4, 1, HIDDEN)), _full((4, 1, HIDDEN))],
        [y_shape, stats_shape, stats_shape],
        (*scalars, xi, xj, xk, geo_s, Wa, Wb, Wc, Wg, b0[:, None, :]),
    )

    for layer in range(1, 4):
        y, s, q = _tc_call(
            _mid_body, nb,
            [_rows(HIDDEN), _full((4, 1, HIDDEN)), _full((4, 1, HIDDEN)),
             _full((4, HIDDEN, HIDDEN)), _full((4, 1, HIDDEN)),
             _full((4, 1, HIDDEN)), _full((4, 1, HIDDEN))],
            [_rows(HIDDEN), _full((4, 1, HIDDEN)), _full((4, 1, HIDDEN))],
            [y_shape, stats_shape, stats_shape],
            (*scalars, y, s, q, Wh[:, layer - 1], bh[:, layer - 1, :, None].transpose(0, 2, 1),
             gamma[:, layer - 1][:, None, :], beta[:, layer - 1][:, None, :]),
        )

    z = _tc_call(
        _fin_body, nb,
        [_rows(HIDDEN), _full((4, 1, HIDDEN)), _full((4, 1, HIDDEN)),
         _full((4, 1, HIDDEN)), _full((4, 1, HIDDEN)), _full((4, 1, HIDDEN))],
        [_rows(HIDDEN)],
        [y_shape],
        (*scalars, y, s, q, gamma[:, 3][:, None, :], beta[:, 3][:, None, :], att_b),
    )[0]

    zeros_init = jnp.zeros((NACC, HIDDEN), jnp.float32)
    partials = _sc_scatter(z, i.astype(jnp.int32), slot, zeros_init)

    out = pl.pallas_call(
        _add_body,
        grid=(N_NODES // 400,),
        in_specs=[pl.BlockSpec((400, HIDDEN), lambda m: (m, 0)),
                  pl.BlockSpec((400, HIDDEN), lambda m: (m, 0))],
        out_specs=pl.BlockSpec((400, HIDDEN), lambda m: (m, 0)),
        out_shape=jax.ShapeDtypeStruct((N_NODES, HIDDEN), jnp.float32),
    )(partials[0], partials[1])
    return out


# speedup vs baseline: 2.3341x; 1.0902x over previous
import functools

import jax
import jax.numpy as jnp
from jax import lax
from jax.experimental import pallas as pl
from jax.experimental.pallas import tpu as pltpu
from jax.experimental.pallas import tpu_sc as plsc

N_NODES = 10000
NACC = 10240
HIDDEN = 128
GEO_PAD = 128
EPS = 1e-5
BLK = 4096
CH = 128
NW = 32


def _sc_redistribute(nf, geo_pad, i_o, j_o, k_o, slot):
    n = i_o.shape[0]
    P = ((n + 4 * BLK) + BLK - 1) // BLK * BLK
    nchunk = n // CH
    steps = (nchunk + NW - 1) // NW
    mesh = plsc.VectorSubcoreMesh(core_axis_name="c", subcore_axis_name="s")

    @functools.partial(
        pl.kernel,
        out_type=[
            jax.ShapeDtypeStruct((P, HIDDEN), jnp.float32),
            jax.ShapeDtypeStruct((P, HIDDEN), jnp.float32),
            jax.ShapeDtypeStruct((P, HIDDEN), jnp.float32),
            jax.ShapeDtypeStruct((P, GEO_PAD), jnp.float32),
        ],
        mesh=mesh,
        scratch_types=[
            pltpu.VMEM((CH,), jnp.int32),
            pltpu.VMEM((CH,), jnp.int32),
            pltpu.VMEM((CH, HIDDEN), jnp.float32),
            pltpu.VMEM((CH, GEO_PAD), jnp.float32),
            pltpu.SemaphoreType.DMA,
        ],
    )
    def k(nf_h, geo_h, i_h, j_h, k_h, slot_h, xi_h, xj_h, xk_h, geo_o,
          idx_v, slot_v, buf_v, gbuf_v, sem):
        wid = lax.axis_index("s") * 2 + lax.axis_index("c")

        def body(t, carry):
            g = wid + t * NW

            @pl.when(g < nchunk)
            def _():
                base = g * CH
                pltpu.sync_copy(slot_h.at[pl.ds(base, CH)], slot_v)
                for ih, oh in ((i_h, xi_h), (j_h, xj_h), (k_h, xk_h)):
                    pltpu.sync_copy(ih.at[pl.ds(base, CH)], idx_v)
                    pltpu.async_copy(nf_h.at[idx_v], buf_v, sem).wait()
                    pltpu.async_copy(buf_v, oh.at[slot_v], sem).wait()
                pltpu.sync_copy(geo_h.at[pl.ds(base, CH)], gbuf_v)
                pltpu.async_copy(gbuf_v, geo_o.at[slot_v], sem).wait()

            return carry

        lax.fori_loop(0, steps, body, 0)

    return k(nf, geo_pad, i_o, j_o, k_o, slot)


def _sc_scatter(z, i_o, slot, zeros_init):
    n = i_o.shape[0]
    nchunk = n // CH
    half0 = (nchunk + 1) // 2
    steps = (half0 + 15) // 16
    rows_per_tile = NACC // 16
    mesh = plsc.VectorSubcoreMesh(core_axis_name="c", subcore_axis_name="s")

    @functools.partial(
        pl.kernel,
        out_type=jax.ShapeDtypeStruct((2, NACC, HIDDEN), jnp.float32),
        mesh=mesh,
        scratch_types=[
            pltpu.VMEM((CH,), jnp.int32),
            pltpu.VMEM((CH,), jnp.int32),
            pltpu.VMEM((CH, HIDDEN), jnp.float32),
            pltpu.VMEM_SHARED((NACC, HIDDEN), jnp.float32),
            pltpu.SemaphoreType.DMA,
        ],
    )
    def k(z_h, i_h, slot_h, zero_h, out_h, idx_v, slot_v, z_v, acc, sem):
        c = lax.axis_index("c")
        s = lax.axis_index("s")

        @pl.when(s == 0)
        def _():
            pltpu.sync_copy(zero_h, acc)

        plsc.subcore_barrier()
        start_c = c * half0
        end_c = half0 + c * (nchunk - half0)

        def body(t, carry):
            g = start_c + s + t * 16

            @pl.when(g < end_c)
            def _():
                base = g * CH
                pltpu.sync_copy(slot_h.at[pl.ds(base, CH)], slot_v)
                pltpu.async_copy(z_h.at[slot_v], z_v, sem).wait()
                pltpu.sync_copy(i_h.at[pl.ds(base, CH)], idx_v)
                pltpu.sync_copy(z_v, acc.at[idx_v], add=True)

            return carry

        lax.fori_loop(0, steps, body, 0)
        plsc.subcore_barrier()
        r0 = s * rows_per_tile
        pltpu.sync_copy(acc.at[pl.ds(r0, rows_per_tile)],
                        out_h.at[c, pl.ds(r0, rows_per_tile)])

    return k(z, i_o, slot, zeros_init)


def _valid_mask(m, br, ve_ref):
    rows = m * BLK + lax.broadcasted_iota(jnp.int32, (BLK, 1), 0)
    return rows < ve_ref[br]


def _accum_stats(m, br, y, v, s_ref, q_ref):
    @pl.when(m == 0)
    def _():
        s_ref[...] = jnp.zeros_like(s_ref)
        q_ref[...] = jnp.zeros_like(q_ref)

    s_ref[br] += jnp.sum(jnp.where(v, y, 0.0), axis=0, keepdims=True)
    q_ref[br] += jnp.sum(jnp.where(v, y * y, 0.0), axis=0, keepdims=True)


def _l0_body(bb_ref, ve_ref, cnt_ref, xi, xj, xk, gg, wa, wb, wc, wg, b0,
             y_ref, s_ref, q_ref):
    m = pl.program_id(0)
    br = bb_ref[m]
    y = jnp.dot(xi[...], wa[br], preferred_element_type=jnp.float32)
    y += jnp.dot(xj[...], wb[br], preferred_element_type=jnp.float32)
    y += jnp.dot(xk[...], wc[br], preferred_element_type=jnp.float32)
    y += jnp.dot(gg[...], wg[br], preferred_element_type=jnp.float32)
    y += b0[br]
    y_ref[...] = y
    _accum_stats(m, br, y, _valid_mask(m, br, ve_ref), s_ref, q_ref)


def _mid_body(bb_ref, ve_ref, cnt_ref, yp, sp, qp, w, bv, ga, be,
              y_ref, s_ref, q_ref):
    m = pl.program_id(0)
    br = bb_ref[m]
    cnt = cnt_ref[br]
    mean = sp[br] / cnt
    var = qp[br] / cnt - mean * mean
    x = (yp[...] - mean) * lax.rsqrt(var + EPS)
    x = x * ga[br] + be[br]
    x = jnp.maximum(x, 0.0)
    y = jnp.dot(x, w[br], preferred_element_type=jnp.float32) + bv[br]
    y_ref[...] = y
    _accum_stats(m, br, y, _valid_mask(m, br, ve_ref), s_ref, q_ref)


def _fin_body(bb_ref, ve_ref, cnt_ref, yp, sp, qp, ga, be, attb, z_ref):
    m = pl.program_id(0)
    br = bb_ref[m]
    cnt = cnt_ref[br]
    mean = sp[br] / cnt
    var = qp[br] / cnt - mean * mean
    x = (yp[...] - mean) * lax.rsqrt(var + EPS)
    x = x * ga[br] + be[br]
    x = jnp.maximum(x, 0.0)
    z_ref[...] = jnp.where(_valid_mask(m, br, ve_ref), x * attb[br], 0.0)


def _full(shape):
    nd = len(shape)
    return pl.BlockSpec(shape, lambda m, *_: (0,) * nd)


def _rows(width):
    return pl.BlockSpec((BLK, width), lambda m, *_: (m, 0))


def _tc_call(body, nb, in_specs, out_specs, out_shape, args):
    grid_spec = pltpu.PrefetchScalarGridSpec(
        num_scalar_prefetch=3,
        grid=(nb,),
        in_specs=in_specs,
        out_specs=out_specs,
    )
    return pl.pallas_call(body, grid_spec=grid_spec, out_shape=out_shape)(*args)


def _add_body(a, b, o):
    o[...] = a[...] + b[...]


def kernel(node_feature, geo_encoding, edge_index_2rd, edx_jk, edx_ij,
           num_edge_inside, att, g, W0, b0, Wh, bh, gamma, beta):
    n = edx_ij.shape[0]
    P = ((n + 4 * BLK) + BLK - 1) // BLK * BLK
    nb = P // BLK

    i = edge_index_2rd[0]
    j = edge_index_2rd[1]
    kk = edge_index_2rd[2]
    br = 2 * (edx_ij >= num_edge_inside).astype(jnp.int32) \
        + (edx_jk >= num_edge_inside).astype(jnp.int32)

    onehot = (br[:, None] == jnp.arange(4)[None, :]).astype(jnp.int32)
    counts = jnp.sum(onehot, axis=0)
    padded = ((counts + BLK - 1) // BLK) * BLK
    pstart = jnp.concatenate([jnp.zeros(1, jnp.int32),
                              jnp.cumsum(padded)[:3].astype(jnp.int32)])
    rank = jnp.sum(jnp.cumsum(onehot, axis=0) * onehot, axis=1) - 1
    slot = (pstart[br] + rank).astype(jnp.int32)
    valid_end = pstart + counts
    block_starts = jnp.arange(nb, dtype=jnp.int32) * BLK
    block_branch = jnp.sum(
        (block_starts[:, None] >= pstart[None, 1:]).astype(jnp.int32), axis=1)
    cnt_f = counts.astype(jnp.float32)

    geo_pad = jnp.pad(geo_encoding, ((0, 0), (0, GEO_PAD - geo_encoding.shape[1])))
    Wa = W0[:, 0:HIDDEN]
    Wb = W0[:, HIDDEN:2 * HIDDEN]
    Wc = W0[:, 2 * HIDDEN:3 * HIDDEN]
    Wg = jnp.pad(W0[:, 3 * HIDDEN:], ((0, 0), (0, GEO_PAD - (W0.shape[1] - 3 * HIDDEN)), (0, 0)))
    att_b = jnp.broadcast_to(att[:, None, None], (4, 1, HIDDEN))

    xi, xj, xk, geo_s = _sc_redistribute(node_feature, geo_pad,
                                         i.astype(jnp.int32),
                                         j.astype(jnp.int32),
                                         kk.astype(jnp.int32), slot)

    stats_shape = jax.ShapeDtypeStruct((4, 1, HIDDEN), jnp.float32)
    y_shape = jax.ShapeDtypeStruct((P, HIDDEN), jnp.float32)
    scalars = (block_branch, valid_end, cnt_f)

    y, s, q = _tc_call(
        _l0_body, nb,
        [_rows(HIDDEN), _rows(HIDDEN), _rows(HIDDEN), _rows(GEO_PAD),
         _full((4, HIDDEN, HIDDEN)), _full((4, HIDDEN, HIDDEN)),
         _full((4, HIDDEN, HIDDEN)), _full((4, GEO_PAD, HIDDEN)),
         _full((4, 1, HIDDEN))],
        [_rows(HIDDEN), _full((4, 1, HIDDEN)), _full((4, 1, HIDDEN))],
        [y_shape, stats_shape, stats_shape],
        (*scalars, xi, xj, xk, geo_s, Wa, Wb, Wc, Wg, b0[:, None, :]),
    )

    for layer in range(1, 4):
        y, s, q = _tc_call(
            _mid_body, nb,
            [_rows(HIDDEN), _full((4, 1, HIDDEN)), _full((4, 1, HIDDEN)),
             _full((4, HIDDEN, HIDDEN)), _full((4, 1, HIDDEN)),
             _full((4, 1, HIDDEN)), _full((4, 1, HIDDEN))],
            [_rows(HIDDEN), _full((4, 1, HIDDEN)), _full((4, 1, HIDDEN))],
            [y_shape, stats_shape, stats_shape],
            (*scalars, y, s, q, Wh[:, layer - 1], bh[:, layer - 1, :, None].transpose(0, 2, 1),
             gamma[:, layer - 1][:, None, :], beta[:, layer - 1][:, None, :]),
        )

    z = _tc_call(
        _fin_body, nb,
        [_rows(HIDDEN), _full((4, 1, HIDDEN)), _full((4, 1, HIDDEN)),
         _full((4, 1, HIDDEN)), _full((4, 1, HIDDEN)), _full((4, 1, HIDDEN))],
        [_rows(HIDDEN)],
        [y_shape],
        (*scalars, y, s, q, gamma[:, 3][:, None, :], beta[:, 3][:, None, :], att_b),
    )[0]

    zeros_init = jnp.zeros((NACC, HIDDEN), jnp.float32)
    partials = _sc_scatter(z, i.astype(jnp.int32), slot, zeros_init)

    out = pl.pallas_call(
        _add_body,
        grid=(N_NODES // 400,),
        in_specs=[pl.BlockSpec((400, HIDDEN), lambda m: (m, 0)),
                  pl.BlockSpec((400, HIDDEN), lambda m: (m, 0))],
        out_specs=pl.BlockSpec((400, HIDDEN), lambda m: (m, 0)),
        out_shape=jax.ShapeDtypeStruct((N_NODES, HIDDEN), jnp.float32),
    )(partials[0], partials[1])
    return out


# speedup vs baseline: 2.7066x; 1.1596x over previous
import functools

import jax
import jax.numpy as jnp
from jax import lax
from jax.experimental import pallas as pl
from jax.experimental.pallas import tpu as pltpu
from jax.experimental.pallas import tpu_sc as plsc

N_NODES = 10000
NACC = 10240
HIDDEN = 128
GEO_PAD = 128
EPS = 1e-5
BLK = 4096
CH = 320
NW = 32


def _sc_redistribute(nf, geo_pad, i_o, j_o, k_o, slot):
    n = i_o.shape[0]
    P = ((n + 4 * BLK) + BLK - 1) // BLK * BLK
    nchunk = n // CH
    steps = (nchunk + NW - 1) // NW
    mesh = plsc.VectorSubcoreMesh(core_axis_name="c", subcore_axis_name="s")

    @functools.partial(
        pl.kernel,
        out_type=[
            jax.ShapeDtypeStruct((P, HIDDEN), jnp.float32),
            jax.ShapeDtypeStruct((P, HIDDEN), jnp.float32),
            jax.ShapeDtypeStruct((P, HIDDEN), jnp.float32),
            jax.ShapeDtypeStruct((P, GEO_PAD), jnp.float32),
        ],
        mesh=mesh,
        scratch_types=[
            pltpu.VMEM((CH,), jnp.int32),
            pltpu.VMEM((CH,), jnp.int32),
            pltpu.VMEM((CH, HIDDEN), jnp.float32),
            pltpu.VMEM((CH, GEO_PAD), jnp.float32),
            pltpu.SemaphoreType.DMA,
        ],
    )
    def k(nf_h, geo_h, i_h, j_h, k_h, slot_h, xi_h, xj_h, xk_h, geo_o,
          idx_v, slot_v, buf_v, gbuf_v, sem):
        wid = lax.axis_index("s") * 2 + lax.axis_index("c")

        def body(t, carry):
            g = wid + t * NW

            @pl.when(g < nchunk)
            def _():
                base = g * CH
                pltpu.sync_copy(slot_h.at[pl.ds(base, CH)], slot_v)
                for ih, oh in ((i_h, xi_h), (j_h, xj_h), (k_h, xk_h)):
                    pltpu.sync_copy(ih.at[pl.ds(base, CH)], idx_v)
                    pltpu.async_copy(nf_h.at[idx_v], buf_v, sem).wait()
                    pltpu.async_copy(buf_v, oh.at[slot_v], sem).wait()
                pltpu.sync_copy(geo_h.at[pl.ds(base, CH)], gbuf_v)
                pltpu.async_copy(gbuf_v, geo_o.at[slot_v], sem).wait()

            return carry

        lax.fori_loop(0, steps, body, 0)

    return k(nf, geo_pad, i_o, j_o, k_o, slot)


def _sc_scatter(z, i_o, slot, zeros_init):
    n = i_o.shape[0]
    nchunk = n // CH
    half0 = (nchunk + 1) // 2
    steps = (half0 + 15) // 16
    rows_per_tile = NACC // 16
    mesh = plsc.VectorSubcoreMesh(core_axis_name="c", subcore_axis_name="s")

    @functools.partial(
        pl.kernel,
        out_type=jax.ShapeDtypeStruct((2, NACC, HIDDEN), jnp.float32),
        mesh=mesh,
        scratch_types=[
            pltpu.VMEM((CH,), jnp.int32),
            pltpu.VMEM((CH,), jnp.int32),
            pltpu.VMEM((CH, HIDDEN), jnp.float32),
            pltpu.VMEM_SHARED((NACC, HIDDEN), jnp.float32),
            pltpu.SemaphoreType.DMA,
        ],
    )
    def k(z_h, i_h, slot_h, zero_h, out_h, idx_v, slot_v, z_v, acc, sem):
        c = lax.axis_index("c")
        s = lax.axis_index("s")

        @pl.when(s == 0)
        def _():
            pltpu.sync_copy(zero_h, acc)

        plsc.subcore_barrier()
        start_c = c * half0
        end_c = half0 + c * (nchunk - half0)

        def body(t, carry):
            g = start_c + s + t * 16

            @pl.when(g < end_c)
            def _():
                base = g * CH
                pltpu.sync_copy(slot_h.at[pl.ds(base, CH)], slot_v)
                pltpu.async_copy(z_h.at[slot_v], z_v, sem).wait()
                pltpu.sync_copy(i_h.at[pl.ds(base, CH)], idx_v)
                pltpu.sync_copy(z_v, acc.at[idx_v], add=True)

            return carry

        lax.fori_loop(0, steps, body, 0)
        plsc.subcore_barrier()
        r0 = s * rows_per_tile
        pltpu.sync_copy(acc.at[pl.ds(r0, rows_per_tile)],
                        out_h.at[c, pl.ds(r0, rows_per_tile)])

    return k(z, i_o, slot, zeros_init)


def _valid_mask(m, br, ve_ref):
    rows = m * BLK + lax.broadcasted_iota(jnp.int32, (BLK, 1), 0)
    return rows < ve_ref[br]


def _accum_stats(m, br, y, v, s_ref, q_ref):
    @pl.when(m == 0)
    def _():
        s_ref[...] = jnp.zeros_like(s_ref)
        q_ref[...] = jnp.zeros_like(q_ref)

    s_ref[br] += jnp.sum(jnp.where(v, y, 0.0), axis=0, keepdims=True)
    q_ref[br] += jnp.sum(jnp.where(v, y * y, 0.0), axis=0, keepdims=True)


def _l0_body(bb_ref, ve_ref, cnt_ref, xi, xj, xk, gg, wa, wb, wc, wg, b0,
             y_ref, s_ref, q_ref):
    m = pl.program_id(0)
    br = bb_ref[m]
    y = jnp.dot(xi[...], wa[br], preferred_element_type=jnp.float32)
    y += jnp.dot(xj[...], wb[br], preferred_element_type=jnp.float32)
    y += jnp.dot(xk[...], wc[br], preferred_element_type=jnp.float32)
    y += jnp.dot(gg[...], wg[br], preferred_element_type=jnp.float32)
    y += b0[br]
    y_ref[...] = y
    _accum_stats(m, br, y, _valid_mask(m, br, ve_ref), s_ref, q_ref)


def _mid_body(bb_ref, ve_ref, cnt_ref, yp, sp, qp, w, bv, ga, be,
              y_ref, s_ref, q_ref):
    m = pl.program_id(0)
    br = bb_ref[m]
    cnt = cnt_ref[br]
    mean = sp[br] / cnt
    var = qp[br] / cnt - mean * mean
    x = (yp[...] - mean) * lax.rsqrt(var + EPS)
    x = x * ga[br] + be[br]
    x = jnp.maximum(x, 0.0)
    y = jnp.dot(x, w[br], preferred_element_type=jnp.float32) + bv[br]
    y_ref[...] = y
    _accum_stats(m, br, y, _valid_mask(m, br, ve_ref), s_ref, q_ref)


def _fin_body(bb_ref, ve_ref, cnt_ref, yp, sp, qp, ga, be, attb, z_ref):
    m = pl.program_id(0)
    br = bb_ref[m]
    cnt = cnt_ref[br]
    mean = sp[br] / cnt
    var = qp[br] / cnt - mean * mean
    x = (yp[...] - mean) * lax.rsqrt(var + EPS)
    x = x * ga[br] + be[br]
    x = jnp.maximum(x, 0.0)
    z_ref[...] = jnp.where(_valid_mask(m, br, ve_ref), x * attb[br], 0.0)


def _full(shape):
    nd = len(shape)
    return pl.BlockSpec(shape, lambda m, *_: (0,) * nd)


def _rows(width):
    return pl.BlockSpec((BLK, width), lambda m, *_: (m, 0))


def _tc_call(body, nb, in_specs, out_specs, out_shape, args):
    grid_spec = pltpu.PrefetchScalarGridSpec(
        num_scalar_prefetch=3,
        grid=(nb,),
        in_specs=in_specs,
        out_specs=out_specs,
    )
    return pl.pallas_call(body, grid_spec=grid_spec, out_shape=out_shape)(*args)


def _add_body(a, b, o):
    o[...] = a[...] + b[...]


def kernel(node_feature, geo_encoding, edge_index_2rd, edx_jk, edx_ij,
           num_edge_inside, att, g, W0, b0, Wh, bh, gamma, beta):
    n = edx_ij.shape[0]
    P = ((n + 4 * BLK) + BLK - 1) // BLK * BLK
    nb = P // BLK

    i = edge_index_2rd[0]
    j = edge_index_2rd[1]
    kk = edge_index_2rd[2]
    br = 2 * (edx_ij >= num_edge_inside).astype(jnp.int32) \
        + (edx_jk >= num_edge_inside).astype(jnp.int32)

    onehot = (br[:, None] == jnp.arange(4)[None, :]).astype(jnp.int32)
    counts = jnp.sum(onehot, axis=0)
    padded = ((counts + BLK - 1) // BLK) * BLK
    pstart = jnp.concatenate([jnp.zeros(1, jnp.int32),
                              jnp.cumsum(padded)[:3].astype(jnp.int32)])
    rank = jnp.sum(jnp.cumsum(onehot, axis=0) * onehot, axis=1) - 1
    slot = (pstart[br] + rank).astype(jnp.int32)
    valid_end = pstart + counts
    block_starts = jnp.arange(nb, dtype=jnp.int32) * BLK
    block_branch = jnp.sum(
        (block_starts[:, None] >= pstart[None, 1:]).astype(jnp.int32), axis=1)
    cnt_f = counts.astype(jnp.float32)

    geo_pad = jnp.pad(geo_encoding, ((0, 0), (0, GEO_PAD - geo_encoding.shape[1])))
    Wa = W0[:, 0:HIDDEN]
    Wb = W0[:, HIDDEN:2 * HIDDEN]
    Wc = W0[:, 2 * HIDDEN:3 * HIDDEN]
    Wg = jnp.pad(W0[:, 3 * HIDDEN:], ((0, 0), (0, GEO_PAD - (W0.shape[1] - 3 * HIDDEN)), (0, 0)))
    att_b = jnp.broadcast_to(att[:, None, None], (4, 1, HIDDEN))

    xi, xj, xk, geo_s = _sc_redistribute(node_feature, geo_pad,
                                         i.astype(jnp.int32),
                                         j.astype(jnp.int32),
                                         kk.astype(jnp.int32), slot)

    stats_shape = jax.ShapeDtypeStruct((4, 1, HIDDEN), jnp.float32)
    y_shape = jax.ShapeDtypeStruct((P, HIDDEN), jnp.float32)
    scalars = (block_branch, valid_end, cnt_f)

    y, s, q = _tc_call(
        _l0_body, nb,
        [_rows(HIDDEN), _rows(HIDDEN), _rows(HIDDEN), _rows(GEO_PAD),
         _full((4, HIDDEN, HIDDEN)), _full((4, HIDDEN, HIDDEN)),
         _full((4, HIDDEN, HIDDEN)), _full((4, GEO_PAD, HIDDEN)),
         _full((4, 1, HIDDEN))],
        [_rows(HIDDEN), _full((4, 1, HIDDEN)), _full((4, 1, HIDDEN))],
        [y_shape, stats_shape, stats_shape],
        (*scalars, xi, xj, xk, geo_s, Wa, Wb, Wc, Wg, b0[:, None, :]),
    )

    for layer in range(1, 4):
        y, s, q = _tc_call(
            _mid_body, nb,
            [_rows(HIDDEN), _full((4, 1, HIDDEN)), _full((4, 1, HIDDEN)),
             _full((4, HIDDEN, HIDDEN)), _full((4, 1, HIDDEN)),
             _full((4, 1, HIDDEN)), _full((4, 1, HIDDEN))],
            [_rows(HIDDEN), _full((4, 1, HIDDEN)), _full((4, 1, HIDDEN))],
            [y_shape, stats_shape, stats_shape],
            (*scalars, y, s, q, Wh[:, layer - 1], bh[:, layer - 1, :, None].transpose(0, 2, 1),
             gamma[:, layer - 1][:, None, :], beta[:, layer - 1][:, None, :]),
        )

    z = _tc_call(
        _fin_body, nb,
        [_rows(HIDDEN), _full((4, 1, HIDDEN)), _full((4, 1, HIDDEN)),
         _full((4, 1, HIDDEN)), _full((4, 1, HIDDEN)), _full((4, 1, HIDDEN))],
        [_rows(HIDDEN)],
        [y_shape],
        (*scalars, y, s, q, gamma[:, 3][:, None, :], beta[:, 3][:, None, :], att_b),
    )[0]

    zeros_init = jnp.zeros((NACC, HIDDEN), jnp.float32)
    partials = _sc_scatter(z, i.astype(jnp.int32), slot, zeros_init)

    out = pl.pallas_call(
        _add_body,
        grid=(N_NODES // 400,),
        in_specs=[pl.BlockSpec((400, HIDDEN), lambda m: (m, 0)),
                  pl.BlockSpec((400, HIDDEN), lambda m: (m, 0))],
        out_specs=pl.BlockSpec((400, HIDDEN), lambda m: (m, 0)),
        out_shape=jax.ShapeDtypeStruct((N_NODES, HIDDEN), jnp.float32),
    )(partials[0], partials[1])
    return out


# speedup vs baseline: 2.7624x; 1.0206x over previous
import functools

import jax
import jax.numpy as jnp
from jax import lax
from jax.experimental import pallas as pl
from jax.experimental.pallas import tpu as pltpu
from jax.experimental.pallas import tpu_sc as plsc

N_NODES = 10000
NACC = 10240
HIDDEN = 128
GEO_PAD = 128
EPS = 1e-5
BLK = 4096
CH = 320
NW = 32


def _sc_redistribute(nf, geo_pad, i_o, j_o, k_o, slot):
    n = i_o.shape[0]
    P = ((n + 4 * BLK) + BLK - 1) // BLK * BLK
    nchunk = n // CH
    steps = (nchunk + NW - 1) // NW
    mesh = plsc.VectorSubcoreMesh(core_axis_name="c", subcore_axis_name="s")

    @functools.partial(
        pl.kernel,
        out_type=[
            jax.ShapeDtypeStruct((P, HIDDEN), jnp.float32),
            jax.ShapeDtypeStruct((P, HIDDEN), jnp.float32),
            jax.ShapeDtypeStruct((P, HIDDEN), jnp.float32),
            jax.ShapeDtypeStruct((P, GEO_PAD), jnp.float32),
        ],
        mesh=mesh,
        scratch_types=[
            pltpu.VMEM((CH,), jnp.int32),
            pltpu.VMEM((CH,), jnp.int32),
            pltpu.VMEM((CH,), jnp.int32),
            pltpu.VMEM((CH,), jnp.int32),
            pltpu.VMEM((CH, HIDDEN), jnp.float32),
            pltpu.VMEM((CH, HIDDEN), jnp.float32),
            pltpu.SemaphoreType.DMA,
            pltpu.SemaphoreType.DMA,
            pltpu.SemaphoreType.DMA,
            pltpu.SemaphoreType.DMA,
        ],
    )
    def k(nf_h, geo_h, i_h, j_h, k_h, slot_h, xi_h, xj_h, xk_h, geo_o,
          ii_v, jj_v, kk_v, slot_v, buf0, buf1, semA, semB, semC, semD):
        wid = lax.axis_index("s") * 2 + lax.axis_index("c")

        def body(t, carry):
            g = wid + t * NW

            @pl.when(g < nchunk)
            def _():
                base = g * CH
                pltpu.sync_copy(slot_h.at[pl.ds(base, CH)], slot_v)
                pltpu.sync_copy(i_h.at[pl.ds(base, CH)], ii_v)
                pltpu.sync_copy(j_h.at[pl.ds(base, CH)], jj_v)
                pltpu.sync_copy(k_h.at[pl.ds(base, CH)], kk_v)
                hg0 = pltpu.async_copy(nf_h.at[ii_v], buf0, semA)
                hg1 = pltpu.async_copy(nf_h.at[jj_v], buf1, semB)
                hg0.wait()
                hs0 = pltpu.async_copy(buf0, xi_h.at[slot_v], semC)
                hg1.wait()
                hs1 = pltpu.async_copy(buf1, xj_h.at[slot_v], semD)
                hs0.wait()
                hg2 = pltpu.async_copy(nf_h.at[kk_v], buf0, semA)
                hs1.wait()
                hg3 = pltpu.async_copy(geo_h.at[pl.ds(base, CH)], buf1, semB)
                hg2.wait()
                hs2 = pltpu.async_copy(buf0, xk_h.at[slot_v], semC)
                hg3.wait()
                hs3 = pltpu.async_copy(buf1, geo_o.at[slot_v], semD)
                hs2.wait()
                hs3.wait()

            return carry

        lax.fori_loop(0, steps, body, 0)

    return k(nf, geo_pad, i_o, j_o, k_o, slot)


def _sc_scatter(z, i_o, slot, zeros_init):
    n = i_o.shape[0]
    nchunk = n // CH
    half0 = (nchunk + 1) // 2
    steps = (half0 + 15) // 16
    rows_per_tile = NACC // 16
    mesh = plsc.VectorSubcoreMesh(core_axis_name="c", subcore_axis_name="s")

    @functools.partial(
        pl.kernel,
        out_type=jax.ShapeDtypeStruct((2, NACC, HIDDEN), jnp.float32),
        mesh=mesh,
        scratch_types=[
            pltpu.VMEM((CH,), jnp.int32),
            pltpu.VMEM((CH,), jnp.int32),
            pltpu.VMEM((CH, HIDDEN), jnp.float32),
            pltpu.VMEM_SHARED((NACC, HIDDEN), jnp.float32),
            pltpu.SemaphoreType.DMA,
        ],
    )
    def k(z_h, i_h, slot_h, zero_h, out_h, idx_v, slot_v, z_v, acc, sem):
        c = lax.axis_index("c")
        s = lax.axis_index("s")

        @pl.when(s == 0)
        def _():
            pltpu.sync_copy(zero_h, acc)

        plsc.subcore_barrier()
        start_c = c * half0
        end_c = half0 + c * (nchunk - half0)

        def body(t, carry):
            g = start_c + s + t * 16

            @pl.when(g < end_c)
            def _():
                base = g * CH
                pltpu.sync_copy(slot_h.at[pl.ds(base, CH)], slot_v)
                pltpu.async_copy(z_h.at[slot_v], z_v, sem).wait()
                pltpu.sync_copy(i_h.at[pl.ds(base, CH)], idx_v)
                pltpu.sync_copy(z_v, acc.at[idx_v], add=True)

            return carry

        lax.fori_loop(0, steps, body, 0)
        plsc.subcore_barrier()
        r0 = s * rows_per_tile
        pltpu.sync_copy(acc.at[pl.ds(r0, rows_per_tile)],
                        out_h.at[c, pl.ds(r0, rows_per_tile)])

    return k(z, i_o, slot, zeros_init)


def _valid_mask(m, br, ve_ref):
    rows = m * BLK + lax.broadcasted_iota(jnp.int32, (BLK, 1), 0)
    return rows < ve_ref[br]


def _accum_stats(m, br, y, v, s_ref, q_ref):
    @pl.when(m == 0)
    def _():
        s_ref[...] = jnp.zeros_like(s_ref)
        q_ref[...] = jnp.zeros_like(q_ref)

    s_ref[br] += jnp.sum(jnp.where(v, y, 0.0), axis=0, keepdims=True)
    q_ref[br] += jnp.sum(jnp.where(v, y * y, 0.0), axis=0, keepdims=True)


def _l0_body(bb_ref, ve_ref, cnt_ref, xi, xj, xk, gg, wa, wb, wc, wg, b0,
             y_ref, s_ref, q_ref):
    m = pl.program_id(0)
    br = bb_ref[m]
    y = jnp.dot(xi[...], wa[br], preferred_element_type=jnp.float32)
    y += jnp.dot(xj[...], wb[br], preferred_element_type=jnp.float32)
    y += jnp.dot(xk[...], wc[br], preferred_element_type=jnp.float32)
    y += jnp.dot(gg[...], wg[br], preferred_element_type=jnp.float32)
    y += b0[br]
    y_ref[...] = y
    _accum_stats(m, br, y, _valid_mask(m, br, ve_ref), s_ref, q_ref)


def _mid_body(bb_ref, ve_ref, cnt_ref, yp, sp, qp, w, bv, ga, be,
              y_ref, s_ref, q_ref):
    m = pl.program_id(0)
    br = bb_ref[m]
    cnt = cnt_ref[br]
    mean = sp[br] / cnt
    var = qp[br] / cnt - mean * mean
    x = (yp[...] - mean) * lax.rsqrt(var + EPS)
    x = x * ga[br] + be[br]
    x = jnp.maximum(x, 0.0)
    y = jnp.dot(x, w[br], preferred_element_type=jnp.float32) + bv[br]
    y_ref[...] = y
    _accum_stats(m, br, y, _valid_mask(m, br, ve_ref), s_ref, q_ref)


def _fin_body(bb_ref, ve_ref, cnt_ref, yp, sp, qp, ga, be, attb, z_ref):
    m = pl.program_id(0)
    br = bb_ref[m]
    cnt = cnt_ref[br]
    mean = sp[br] / cnt
    var = qp[br] / cnt - mean * mean
    x = (yp[...] - mean) * lax.rsqrt(var + EPS)
    x = x * ga[br] + be[br]
    x = jnp.maximum(x, 0.0)
    z_ref[...] = jnp.where(_valid_mask(m, br, ve_ref), x * attb[br], 0.0)


def _full(shape):
    nd = len(shape)
    return pl.BlockSpec(shape, lambda m, *_: (0,) * nd)


def _rows(width):
    return pl.BlockSpec((BLK, width), lambda m, *_: (m, 0))


def _tc_call(body, nb, in_specs, out_specs, out_shape, args):
    grid_spec = pltpu.PrefetchScalarGridSpec(
        num_scalar_prefetch=3,
        grid=(nb,),
        in_specs=in_specs,
        out_specs=out_specs,
    )
    return pl.pallas_call(body, grid_spec=grid_spec, out_shape=out_shape)(*args)


def _add_body(a, b, o):
    o[...] = a[...] + b[...]


def kernel(node_feature, geo_encoding, edge_index_2rd, edx_jk, edx_ij,
           num_edge_inside, att, g, W0, b0, Wh, bh, gamma, beta):
    n = edx_ij.shape[0]
    P = ((n + 4 * BLK) + BLK - 1) // BLK * BLK
    nb = P // BLK

    i = edge_index_2rd[0]
    j = edge_index_2rd[1]
    kk = edge_index_2rd[2]
    br = 2 * (edx_ij >= num_edge_inside).astype(jnp.int32) \
        + (edx_jk >= num_edge_inside).astype(jnp.int32)

    onehot = (br[:, None] == jnp.arange(4)[None, :]).astype(jnp.int32)
    counts = jnp.sum(onehot, axis=0)
    padded = ((counts + BLK - 1) // BLK) * BLK
    pstart = jnp.concatenate([jnp.zeros(1, jnp.int32),
                              jnp.cumsum(padded)[:3].astype(jnp.int32)])
    rank = jnp.sum(jnp.cumsum(onehot, axis=0) * onehot, axis=1) - 1
    slot = (pstart[br] + rank).astype(jnp.int32)
    valid_end = pstart + counts
    block_starts = jnp.arange(nb, dtype=jnp.int32) * BLK
    block_branch = jnp.sum(
        (block_starts[:, None] >= pstart[None, 1:]).astype(jnp.int32), axis=1)
    cnt_f = counts.astype(jnp.float32)

    geo_pad = jnp.pad(geo_encoding, ((0, 0), (0, GEO_PAD - geo_encoding.shape[1])))
    Wa = W0[:, 0:HIDDEN]
    Wb = W0[:, HIDDEN:2 * HIDDEN]
    Wc = W0[:, 2 * HIDDEN:3 * HIDDEN]
    Wg = jnp.pad(W0[:, 3 * HIDDEN:], ((0, 0), (0, GEO_PAD - (W0.shape[1] - 3 * HIDDEN)), (0, 0)))
    att_b = jnp.broadcast_to(att[:, None, None], (4, 1, HIDDEN))

    xi, xj, xk, geo_s = _sc_redistribute(node_feature, geo_pad,
                                         i.astype(jnp.int32),
                                         j.astype(jnp.int32),
                                         kk.astype(jnp.int32), slot)

    stats_shape = jax.ShapeDtypeStruct((4, 1, HIDDEN), jnp.float32)
    y_shape = jax.ShapeDtypeStruct((P, HIDDEN), jnp.float32)
    scalars = (block_branch, valid_end, cnt_f)

    y, s, q = _tc_call(
        _l0_body, nb,
        [_rows(HIDDEN), _rows(HIDDEN), _rows(HIDDEN), _rows(GEO_PAD),
         _full((4, HIDDEN, HIDDEN)), _full((4, HIDDEN, HIDDEN)),
         _full((4, HIDDEN, HIDDEN)), _full((4, GEO_PAD, HIDDEN)),
         _full((4, 1, HIDDEN))],
        [_rows(HIDDEN), _full((4, 1, HIDDEN)), _full((4, 1, HIDDEN))],
        [y_shape, stats_shape, stats_shape],
        (*scalars, xi, xj, xk, geo_s, Wa, Wb, Wc, Wg, b0[:, None, :]),
    )

    for layer in range(1, 4):
        y, s, q = _tc_call(
            _mid_body, nb,
            [_rows(HIDDEN), _full((4, 1, HIDDEN)), _full((4, 1, HIDDEN)),
             _full((4, HIDDEN, HIDDEN)), _full((4, 1, HIDDEN)),
             _full((4, 1, HIDDEN)), _full((4, 1, HIDDEN))],
            [_rows(HIDDEN), _full((4, 1, HIDDEN)), _full((4, 1, HIDDEN))],
            [y_shape, stats_shape, stats_shape],
            (*scalars, y, s, q, Wh[:, layer - 1], bh[:, layer - 1, :, None].transpose(0, 2, 1),
             gamma[:, layer - 1][:, None, :], beta[:, layer - 1][:, None, :]),
        )

    z = _tc_call(
        _fin_body, nb,
        [_rows(HIDDEN), _full((4, 1, HIDDEN)), _full((4, 1, HIDDEN)),
         _full((4, 1, HIDDEN)), _full((4, 1, HIDDEN)), _full((4, 1, HIDDEN))],
        [_rows(HIDDEN)],
        [y_shape],
        (*scalars, y, s, q, gamma[:, 3][:, None, :], beta[:, 3][:, None, :], att_b),
    )[0]

    zeros_init = jnp.zeros((NACC, HIDDEN), jnp.float32)
    partials = _sc_scatter(z, i.astype(jnp.int32), slot, zeros_init)

    out = pl.pallas_call(
        _add_body,
        grid=(N_NODES // 400,),
        in_specs=[pl.BlockSpec((400, HIDDEN), lambda m: (m, 0)),
                  pl.BlockSpec((400, HIDDEN), lambda m: (m, 0))],
        out_specs=pl.BlockSpec((400, HIDDEN), lambda m: (m, 0)),
        out_shape=jax.ShapeDtypeStruct((N_NODES, HIDDEN), jnp.float32),
    )(partials[0], partials[1])
    return out
